# Initial kernel scaffold; baseline (speedup 1.0000x reference)
import jax
import jax.numpy as jnp
from jax.experimental import pallas as pl


def kernel(x, edge_index, W, b):
    raise NotImplementedError("write your pallas kernel here")


# speedup vs baseline: 3.2752x; 3.2752x over previous
import functools

import jax
import jax.numpy as jnp
from jax import lax
from jax.experimental import pallas as pl
from jax.experimental.pallas import tpu as pltpu
from jax.experimental.pallas import tpu_sc as plsc

N = 10000
E = 320000
DF = 128
SM = 0.5

NP = 10240
EP = 327680
CH = EP // 128
NT = 32
CPT = CH // NT
IB = 16
NB = CPT // IB
RPT = NP // 16
TRASH = N

_MESH = plsc.VectorSubcoreMesh(core_axis_name="c", subcore_axis_name="s")


def _deg_body(col_hbm, ones_hbm, zeros_hbm, out_hbm, idxv, onesv, accum):
    c = lax.axis_index("c")
    s = lax.axis_index("s")
    w = c * 16 + s
    pltpu.sync_copy(col_hbm.at[pl.ds(w * CPT, CPT)], idxv)
    pltpu.sync_copy(ones_hbm, onesv)
    pltpu.sync_copy(zeros_hbm, accum.at[pl.ds(s * RPT, RPT)])
    plsc.subcore_barrier()

    @pl.loop(0, CPT)
    def _scatter(j):
        pltpu.sync_copy(onesv, accum.at[idxv.at[j]], add=True)

    plsc.subcore_barrier()
    pltpu.sync_copy(accum.at[pl.ds(s * RPT, RPT)],
                    out_hbm.at[pl.ds(c * NP + s * RPT, RPT)])


_deg_call = pl.kernel(
    _deg_body,
    out_type=jax.ShapeDtypeStruct((2 * NP,), jnp.float32),
    mesh=_MESH,
    scratch_types=[
        pltpu.VMEM((CPT, 128), jnp.int32),
        pltpu.VMEM((128,), jnp.float32),
        pltpu.VMEM_SHARED((NP,), jnp.float32),
    ],
)


def _mm_body(x_ref, w_ref, b_ref, d0_ref, d1_ref, sup_ref, feat_ref, dv_ref):
    sup = lax.dot_general(x_ref[...], w_ref[...], (((1,), (1,)), ((), ())),
                          preferred_element_type=jnp.float32) + b_ref[...]
    deg = d0_ref[0] + d1_ref[0] + 1.0
    dv = lax.rsqrt(deg)
    sup_ref[...] = sup
    feat_ref[...] = sup * dv
    dv_ref[...] = dv


def _mm_call(x_p, W, b2, degp):
    blk = 1024
    g = NP // blk
    return pl.pallas_call(
        _mm_body,
        grid=(g,),
        in_specs=[
            pl.BlockSpec((blk, DF), lambda j: (j, 0)),
            pl.BlockSpec((DF, DF), lambda j: (0, 0)),
            pl.BlockSpec((1, DF), lambda j: (0, 0)),
            pl.BlockSpec((1, blk, 1), lambda j: (0, j, 0)),
            pl.BlockSpec((1, blk, 1), lambda j: (1, j, 0)),
        ],
        out_specs=[
            pl.BlockSpec((blk, DF), lambda j: (j, 0)),
            pl.BlockSpec((blk, DF), lambda j: (j, 0)),
            pl.BlockSpec((blk, 1), lambda j: (j, 0)),
        ],
        out_shape=[
            jax.ShapeDtypeStruct((NP, DF), jnp.float32),
            jax.ShapeDtypeStruct((NP, DF), jnp.float32),
            jax.ShapeDtypeStruct((NP, 1), jnp.float32),
        ],
    )(x_p, W, b2, degp, degp)


def _spmm_body(feat_hbm, col_hbm, row_hbm, zeros_hbm, out_hbm,
               colv, rowv, rbuf, accum, sem0, sem1):
    c = lax.axis_index("c")
    s = lax.axis_index("s")
    w = c * 16 + s
    pltpu.sync_copy(zeros_hbm, accum.at[pl.ds(s * RPT, RPT)])
    plsc.subcore_barrier()

    sems = (sem0, sem1)

    @pl.loop(0, NB)
    def _block_loop(bi):
        base = w * CPT + bi * IB
        pltpu.sync_copy(col_hbm.at[pl.ds(base, IB)], colv)
        pltpu.sync_copy(row_hbm.at[pl.ds(base, IB)], rowv)
        pltpu.async_copy(feat_hbm.at[colv.at[0]], rbuf.at[0], sem0)
        pltpu.async_copy(feat_hbm.at[colv.at[1]], rbuf.at[1], sem1)

        @pl.loop(0, IB, step=2)
        def _edge_loop(g):
            for bslot in range(2):
                j = g + bslot
                sem = sems[bslot]
                pltpu.make_async_copy(feat_hbm.at[colv.at[0]],
                                      rbuf.at[bslot], sem).wait()
                pltpu.sync_copy(rbuf.at[bslot], accum.at[rowv.at[j]],
                                add=True)

                @pl.when(j + 2 < IB)
                def _issue():
                    pltpu.async_copy(feat_hbm.at[colv.at[j + 2]],
                                     rbuf.at[bslot], sem)

    plsc.subcore_barrier()
    pltpu.sync_copy(accum.at[pl.ds(s * RPT, RPT)],
                    out_hbm.at[pl.ds(c * NP + s * RPT, RPT)])


_spmm_call = pl.kernel(
    _spmm_body,
    out_type=jax.ShapeDtypeStruct((2 * NP, DF), jnp.float32),
    mesh=_MESH,
    scratch_types=[
        pltpu.VMEM((IB, 128), jnp.int32),
        pltpu.VMEM((IB, 128), jnp.int32),
        pltpu.VMEM((2, 128, DF), jnp.float32),
        pltpu.VMEM_SHARED((NP, DF), jnp.float32),
        pltpu.SemaphoreType.DMA,
        pltpu.SemaphoreType.DMA,
    ],
)


def _epi_body(p0_ref, p1_ref, feat_ref, sup_ref, dv_ref, out_ref):
    agg = (p0_ref[0] + p1_ref[0] + feat_ref[...]) * dv_ref[...]
    out_ref[...] = (agg * SM + sup_ref[...]) * (1.0 / (1.0 + SM))


def _epi_call(partial, feat, support, dv):
    blk = 1000
    g = N // blk
    return pl.pallas_call(
        _epi_body,
        grid=(g,),
        in_specs=[
            pl.BlockSpec((1, blk, DF), lambda j: (0, j, 0)),
            pl.BlockSpec((1, blk, DF), lambda j: (1, j, 0)),
            pl.BlockSpec((blk, DF), lambda j: (j, 0)),
            pl.BlockSpec((blk, DF), lambda j: (j, 0)),
            pl.BlockSpec((blk, 1), lambda j: (j, 0)),
        ],
        out_specs=pl.BlockSpec((blk, DF), lambda j: (j, 0)),
        out_shape=jax.ShapeDtypeStruct((N, DF), jnp.float32),
    )(partial, partial, feat, support, dv)


def kernel(x, edge_index, W, b):
    ei = edge_index.astype(jnp.int32)
    row = ei[0]
    col = ei[1]
    colp = jnp.full((EP,), TRASH, jnp.int32).at[:E].set(col)
    rowp = jnp.full((EP,), TRASH, jnp.int32).at[:E].set(row)
    x_p = jnp.zeros((NP, DF), jnp.float32).at[:N].set(x)

    ones128 = jnp.ones((128,), jnp.float32)
    zeros1 = jnp.zeros((RPT,), jnp.float32)
    zeros128 = jnp.zeros((RPT, DF), jnp.float32)

    degp = _deg_call(colp.reshape(CH, 128), ones128, zeros1).reshape(2, NP, 1)
    support, feat, dv = _mm_call(x_p, W, b.reshape(1, DF), degp)
    partial = _spmm_call(feat, colp.reshape(CH, 128),
                         rowp.reshape(CH, 128), zeros128).reshape(2, NP, DF)
    return _epi_call(partial, feat, support, dv)
